# Initial kernel scaffold; baseline (speedup 1.0000x reference)
import jax
import jax.numpy as jnp
from jax.experimental import pallas as pl


def kernel(x, edge_index, edge_attr, batch, W1, b1, W2, b2, Wend):
    raise NotImplementedError("write your pallas kernel here")


# speedup vs baseline: 25.5680x; 25.5680x over previous
import functools

import jax
import jax.numpy as jnp
from jax import lax
from jax.experimental import pallas as pl
from jax.experimental.pallas import tpu as pltpu
from jax.experimental.pallas import tpu_sc as plsc

NC = 2
NS = 16
L = 16
NW = NC * NS
CHUNK = 128
BLK = 8
BN = 2048


def _mesh():
    return plsc.VectorSubcoreMesh(
        core_axis_name="c", subcore_axis_name="s", num_cores=NC, num_subcores=NS
    )


@functools.lru_cache(maxsize=None)
def _make_degree(n_pad, slabs):
    rpt = n_pad // NS

    @functools.partial(
        pl.kernel,
        out_type=jax.ShapeDtypeStruct((NC, n_pad), jnp.float32),
        mesh=_mesh(),
        scratch_types=[
            pltpu.VMEM((BLK, CHUNK), jnp.int32),
            pltpu.VMEM((BLK, CHUNK), jnp.float32),
            pltpu.VMEM_SHARED((n_pad,), jnp.float32),
        ],
    )
    def deg_kernel(col_h, attr_h, zero_h, out_h, colb, attrb, acc):
        c = lax.axis_index("c")
        s = lax.axis_index("s")
        pltpu.sync_copy(zero_h, acc.at[pl.ds(s * rpt, rpt)])
        plsc.subcore_barrier()
        base = (c * NS + s) * slabs * BLK

        def slab(i, carry):
            st = base + i * BLK
            pltpu.sync_copy(col_h.at[pl.ds(st, BLK)], colb)
            pltpu.sync_copy(attr_h.at[pl.ds(st, BLK)], attrb)
            for j in range(BLK):
                pltpu.sync_copy(attrb.at[j], acc.at[colb.at[j]], add=True)
            return carry

        lax.fori_loop(0, slabs, slab, 0)
        plsc.subcore_barrier()
        pltpu.sync_copy(acc.at[pl.ds(s * rpt, rpt)], out_h.at[c, pl.ds(s * rpt, rpt)])

    return deg_kernel


@functools.lru_cache(maxsize=None)
def _make_hop_planar(n_pad, slabs):
    rpt = n_pad // NS

    @functools.partial(
        pl.kernel,
        out_type=jax.ShapeDtypeStruct((NC * 2, n_pad), jnp.float32),
        mesh=_mesh(),
        scratch_types=[
            pltpu.VMEM((BLK, CHUNK), jnp.int32),
            pltpu.VMEM((BLK, CHUNK), jnp.int32),
            pltpu.VMEM((BLK, CHUNK), jnp.float32),
            pltpu.VMEM((CHUNK,), jnp.float32),
            pltpu.VMEM((CHUNK,), jnp.float32),
            pltpu.VMEM_SHARED((n_pad,), jnp.float32),
            pltpu.VMEM_SHARED((n_pad,), jnp.float32),
            pltpu.VMEM_SHARED((n_pad,), jnp.float32),
            pltpu.VMEM_SHARED((n_pad,), jnp.float32),
            pltpu.SemaphoreType.DMA,
            pltpu.SemaphoreType.DMA,
        ],
    )
    def hop_kernel(t0_h, t1_h, row_h, col_h, attr_h, zero_h, out_h,
                   rowb, colb, attrb, m0, m1, ts0, ts1, ac0, ac1, sem0, sem1):
        c = lax.axis_index("c")
        s = lax.axis_index("s")
        sl = pl.ds(s * rpt, rpt)
        pltpu.sync_copy(zero_h, ac0.at[sl])
        pltpu.sync_copy(zero_h, ac1.at[sl])
        pltpu.sync_copy(t0_h.at[sl], ts0.at[sl])
        pltpu.sync_copy(t1_h.at[sl], ts1.at[sl])
        plsc.subcore_barrier()
        base = (c * NS + s) * slabs * BLK

        def slab(i, carry):
            st = base + i * BLK
            pltpu.sync_copy(row_h.at[pl.ds(st, BLK)], rowb)
            pltpu.sync_copy(col_h.at[pl.ds(st, BLK)], colb)
            pltpu.sync_copy(attr_h.at[pl.ds(st, BLK)], attrb)
            for j in range(BLK):
                cp0 = pltpu.async_copy(ts0.at[rowb.at[j]], m0, sem0)
                cp1 = pltpu.async_copy(ts1.at[rowb.at[j]], m1, sem1)
                cp0.wait()
                cp1.wait()
                for v in range(CHUNK // L):
                    d = pl.ds(v * L, L)
                    a = attrb[j, d]
                    m0[d] = m0[d] * a
                    m1[d] = m1[d] * a
                pltpu.sync_copy(m0, ac0.at[colb.at[j]], add=True)
                pltpu.sync_copy(m1, ac1.at[colb.at[j]], add=True)
            return carry

        lax.fori_loop(0, slabs, slab, 0)
        plsc.subcore_barrier()
        pltpu.sync_copy(ac0.at[sl], out_h.at[c * 2 + 0, sl])
        pltpu.sync_copy(ac1.at[sl], out_h.at[c * 2 + 1, sl])

    return hop_kernel


@functools.lru_cache(maxsize=None)
def _make_hop_rows(n_pad, slabs):
    rpt = n_pad // NS
    f = L

    @functools.partial(
        pl.kernel,
        out_type=jax.ShapeDtypeStruct((NC, n_pad, f), jnp.float32),
        mesh=_mesh(),
        compiler_params=pltpu.CompilerParams(use_tc_tiling_on_sc=False),
        scratch_types=[
            pltpu.VMEM((BLK, CHUNK), jnp.int32),
            pltpu.VMEM((BLK, CHUNK), jnp.int32),
            pltpu.VMEM((BLK, CHUNK), jnp.float32),
            pltpu.VMEM((CHUNK, f), jnp.float32),
            pltpu.VMEM_SHARED((n_pad, f), jnp.float32),
            pltpu.SemaphoreType.DMA,
        ],
    )
    def hop_kernel(t_h, row_h, col_h, attr_h, zero_h, out_h,
                   rowb, colb, attrb, msgb, acc, sem):
        c = lax.axis_index("c")
        s = lax.axis_index("s")
        sl = pl.ds(s * rpt, rpt)
        pltpu.sync_copy(zero_h, acc.at[sl])
        plsc.subcore_barrier()
        base = (c * NS + s) * slabs * BLK
        dnums = lax.GatherDimensionNumbers(
            offset_dims=(), collapsed_slice_dims=(0,), start_index_map=(0,))

        def slab(i, carry):
            st = base + i * BLK
            pltpu.sync_copy(row_h.at[pl.ds(st, BLK)], rowb)
            pltpu.sync_copy(col_h.at[pl.ds(st, BLK)], colb)
            pltpu.sync_copy(attr_h.at[pl.ds(st, BLK)], attrb)
            for j in range(BLK):
                pltpu.async_copy(t_h.at[rowb.at[j]], msgb, sem).wait()

                def grp(g, cg):
                    av = attrb[j, pl.ds(g * L, L)]
                    for t in range(L):
                        e = g * L + t
                        bc = lax.gather(
                            av, jnp.full((L, 1), t, jnp.int32), dnums, (1,),
                            mode=lax.GatherScatterMode.PROMISE_IN_BOUNDS)
                        msgb[e, :] = msgb[e, :] * bc
                    return cg

                lax.fori_loop(0, CHUNK // L, grp, 0)
                pltpu.sync_copy(msgb, acc.at[colb.at[j]], add=True)
            return carry

        lax.fori_loop(0, slabs, slab, 0)
        plsc.subcore_barrier()
        pltpu.sync_copy(acc.at[sl], out_h.at[c, sl])

    return hop_kernel


def _row_spec(bf):
    return pl.BlockSpec((BN, bf), lambda i: (i, 0))


def _full_spec(shape):
    nd = len(shape)
    return pl.BlockSpec(shape, lambda i: (0,) * nd)


def _tc_prep(d0, d1):
    def body(d0r, d1r, o):
        deg = d0r[...] + d1r[...]
        o[...] = jnp.where(deg > 0, lax.rsqrt(jnp.maximum(deg, 1e-30)), 0.0)

    n_pad = d0.shape[0]
    return pl.pallas_call(
        body,
        grid=(n_pad // BN,),
        in_specs=[_row_spec(1), _row_spec(1)],
        out_specs=_row_spec(1),
        out_shape=jax.ShapeDtypeStruct((n_pad, 1), jnp.float32),
    )(d0, d1)


def _tc_init(x0, x1, dis, w, b):
    def body(x0r, x1r, dr, wr, br, acc_o, t0_o, t1_o):
        acc_o[...] = x0r[...] * wr[0:1, :] + x1r[...] * wr[1:2, :] + br[...]
        t0_o[...] = x0r[...] * dr[...]
        t1_o[...] = x1r[...] * dr[...]

    n_pad = x0.shape[0]
    o1 = jax.ShapeDtypeStruct((n_pad, 1), jnp.float32)
    return pl.pallas_call(
        body,
        grid=(n_pad // BN,),
        in_specs=[_row_spec(1), _row_spec(1), _row_spec(1),
                  _full_spec(w.shape), _full_spec(b.shape)],
        out_specs=[_row_spec(16), _row_spec(1), _row_spec(1)],
        out_shape=[jax.ShapeDtypeStruct((n_pad, 16), jnp.float32), o1, o1],
    )(x0, x1, dis, w, b)


def _tc_combine2(q00, q01, q10, q11, dis, acc, w, b):
    def body(a0, a1, b0, b1, dr, ar, wr, br, acc_o, t0_o, t1_o):
        d = dr[...]
        h0 = (a0[...] + b0[...]) * d
        h1 = (a1[...] + b1[...]) * d
        acc_o[...] = ar[...] + h0 * wr[0:1, :] + h1 * wr[1:2, :] + br[...]
        t0_o[...] = h0 * d
        t1_o[...] = h1 * d

    n_pad = q00.shape[0]
    o1 = jax.ShapeDtypeStruct((n_pad, 1), jnp.float32)
    return pl.pallas_call(
        body,
        grid=(n_pad // BN,),
        in_specs=[_row_spec(1)] * 5 + [_row_spec(16),
                  _full_spec(w.shape), _full_spec(b.shape)],
        out_specs=[_row_spec(16), _row_spec(1), _row_spec(1)],
        out_shape=[jax.ShapeDtypeStruct((n_pad, 16), jnp.float32), o1, o1],
    )(q00, q01, q10, q11, dis, acc, w, b)


def _tc_bridge(q00, q01, q10, q11, dis, acc, w14, b14, w20, b20):
    def body(a0, a1, b0, b1, dr, ar, w1r, b1r, w2r, b2r, acc_o, t_o):
        d = dr[...]
        h0 = (a0[...] + b0[...]) * d
        h1 = (a1[...] + b1[...]) * d
        a = jax.nn.relu(ar[...] + h0 * w1r[0:1, :] + h1 * w1r[1:2, :] + b1r[...])
        acc_o[...] = (
            jnp.dot(a, w2r[...], preferred_element_type=jnp.float32) + b2r[...]
        )
        t_o[...] = a * d

    n_pad = q00.shape[0]
    return pl.pallas_call(
        body,
        grid=(n_pad // BN,),
        in_specs=[_row_spec(1)] * 5 + [_row_spec(16),
                  _full_spec(w14.shape), _full_spec(b14.shape),
                  _full_spec(w20.shape), _full_spec(b20.shape)],
        out_specs=[_row_spec(16), _row_spec(16)],
        out_shape=[jax.ShapeDtypeStruct((n_pad, 16), jnp.float32),
                   jax.ShapeDtypeStruct((n_pad, 16), jnp.float32)],
    )(q00, q01, q10, q11, dis, acc, w14, b14, w20, b20)


def _tc_combine16(p0, p1, dis, acc, w, b):
    def body(p0r, p1r, dr, ar, wr, br, acc_o, t_o):
        d = dr[...]
        h = (p0r[...] + p1r[...]) * d
        acc_o[...] = (
            ar[...] + jnp.dot(h, wr[...], preferred_element_type=jnp.float32)
            + br[...]
        )
        t_o[...] = h * d

    n_pad = p0.shape[0]
    return pl.pallas_call(
        body,
        grid=(n_pad // BN,),
        in_specs=[_row_spec(16), _row_spec(16), _row_spec(1), _row_spec(16),
                  _full_spec(w.shape), _full_spec(b.shape)],
        out_specs=[_row_spec(16), _row_spec(16)],
        out_shape=[jax.ShapeDtypeStruct((n_pad, 16), jnp.float32),
                   jax.ShapeDtypeStruct((n_pad, 16), jnp.float32)],
    )(p0, p1, dis, acc, w, b)


def _tc_final(p0, p1, dis, acc, w24, b24, wend):
    def body(p0r, p1r, dr, ar, wr, br, wer, o):
        h = (p0r[...] + p1r[...]) * dr[...]
        a = jax.nn.relu(
            ar[...] + jnp.dot(h, wr[...], preferred_element_type=jnp.float32)
            + br[...]
        )
        o[...] = jax.nn.sigmoid(
            jnp.dot(a, wer[...], preferred_element_type=jnp.float32)
        )

    n_pad = p0.shape[0]
    return pl.pallas_call(
        body,
        grid=(n_pad // BN,),
        in_specs=[_row_spec(16), _row_spec(16), _row_spec(1), _row_spec(16),
                  _full_spec(w24.shape), _full_spec(b24.shape),
                  _full_spec(wend.shape)],
        out_specs=_row_spec(1),
        out_shape=jax.ShapeDtypeStruct((n_pad, 1), jnp.float32),
    )(p0, p1, dis, acc, w24, b24, wend)


def kernel(x, edge_index, edge_attr, batch, W1, b1, W2, b2, Wend):
    n = x.shape[0]
    e = edge_index.shape[1]
    kk = W1.shape[0]

    row = edge_index[0].astype(jnp.int32)
    col = edge_index[1].astype(jnp.int32)
    attr = edge_attr.astype(jnp.float32)

    epw = -(-e // (NW * CHUNK * BLK)) * CHUNK * BLK
    e_pad = NW * epw
    slabs = epw // (CHUNK * BLK)
    pad_n = e_pad - e
    pad_idx = (jnp.arange(pad_n, dtype=jnp.int32) * 37) % n
    row2 = jnp.concatenate([row, pad_idx]).reshape(-1, CHUNK)
    col2 = jnp.concatenate([col, pad_idx]).reshape(-1, CHUNK)
    attr2 = jnp.concatenate(
        [attr, jnp.zeros((pad_n,), jnp.float32)]).reshape(-1, CHUNK)

    n_pad = NS * (-(-n // (NS * CHUNK))) * CHUNK
    rpt = n_pad // NS
    x_p = jnp.zeros((n_pad, x.shape[1]), jnp.float32).at[:n].set(x)

    zero1 = jnp.zeros((rpt,), jnp.float32)
    zero16 = jnp.zeros((rpt, 16), jnp.float32)

    deg_p = _make_degree(n_pad, slabs)(col2, attr2, zero1)
    dis = _tc_prep(deg_p[0].reshape(n_pad, 1), deg_p[1].reshape(n_pad, 1))

    acc, t0, t1 = _tc_init(x_p[:, 0:1], x_p[:, 1:2], dis,
                           W1[0], b1[0].reshape(1, -1))
    hop2 = _make_hop_planar(n_pad, slabs)
    for k in range(1, kk):
        q = hop2(t0.reshape(n_pad), t1.reshape(n_pad),
                 row2, col2, attr2, zero1)
        planes = [q[i].reshape(n_pad, 1) for i in range(4)]
        if k < kk - 1:
            acc, t0, t1 = _tc_combine2(*planes, dis, acc,
                                       W1[k], b1[k].reshape(1, -1))
        else:
            acc, t = _tc_bridge(*planes, dis, acc,
                                W1[k], b1[k].reshape(1, -1),
                                W2[0], b2[0].reshape(1, -1))

    hop16 = _make_hop_rows(n_pad, slabs)
    for k in range(1, kk):
        p = hop16(t, row2, col2, attr2, zero16)
        if k < kk - 1:
            acc, t = _tc_combine16(p[0], p[1], dis, acc,
                                   W2[k], b2[k].reshape(1, -1))
        else:
            out = _tc_final(p[0], p[1], dis, acc,
                            W2[k], b2[k].reshape(1, -1), Wend)

    return out[:n]


# speedup vs baseline: 43.1447x; 1.6874x over previous
import functools

import jax
import jax.numpy as jnp
from jax import lax
from jax.experimental import pallas as pl
from jax.experimental.pallas import tpu as pltpu
from jax.experimental.pallas import tpu_sc as plsc

NC = 2
NS = 16
L = 16
NW = NC * NS
CHUNK = 128
BLK = 16
BN = 2048


def _mesh():
    return plsc.VectorSubcoreMesh(
        core_axis_name="c", subcore_axis_name="s", num_cores=NC, num_subcores=NS
    )


@functools.lru_cache(maxsize=None)
def _make_degree(n_pad, slabs):
    rpt = n_pad // NS

    @functools.partial(
        pl.kernel,
        out_type=jax.ShapeDtypeStruct((NC, n_pad), jnp.float32),
        mesh=_mesh(),
        scratch_types=[
            pltpu.VMEM((BLK, CHUNK), jnp.int32),
            pltpu.VMEM((BLK, CHUNK), jnp.float32),
            pltpu.VMEM_SHARED((n_pad,), jnp.float32),
            pltpu.SemaphoreType.DMA,
        ],
    )
    def deg_kernel(col_h, attr_h, zero_h, out_h, colb, attrb, acc, ssem):
        c = lax.axis_index("c")
        s = lax.axis_index("s")
        pltpu.sync_copy(zero_h, acc.at[pl.ds(s * rpt, rpt)])
        plsc.subcore_barrier()
        base = (c * NS + s) * slabs * BLK

        def slab(i, carry):
            st = base + i * BLK
            pltpu.sync_copy(col_h.at[pl.ds(st, BLK)], colb)
            pltpu.sync_copy(attr_h.at[pl.ds(st, BLK)], attrb)
            descs = [
                pltpu.async_copy(attrb.at[j], acc.at[colb.at[j]], ssem, add=True)
                for j in range(BLK)
            ]
            for d in descs:
                d.wait()
            return carry

        lax.fori_loop(0, slabs, slab, 0)
        plsc.subcore_barrier()
        pltpu.sync_copy(acc.at[pl.ds(s * rpt, rpt)], out_h.at[c, pl.ds(s * rpt, rpt)])

    return deg_kernel


@functools.lru_cache(maxsize=None)
def _make_hop_planar(n_pad, slabs):
    rpt = n_pad // NS

    @functools.partial(
        pl.kernel,
        out_type=jax.ShapeDtypeStruct((NC * 2, n_pad), jnp.float32),
        mesh=_mesh(),
        scratch_types=[
            pltpu.VMEM((BLK, CHUNK), jnp.int32),
            pltpu.VMEM((BLK, CHUNK), jnp.int32),
            pltpu.VMEM((BLK, CHUNK), jnp.float32),
            pltpu.VMEM((BLK, CHUNK), jnp.float32),
            pltpu.VMEM((BLK, CHUNK), jnp.float32),
            pltpu.VMEM_SHARED((n_pad,), jnp.float32),
            pltpu.VMEM_SHARED((n_pad,), jnp.float32),
            pltpu.VMEM_SHARED((n_pad,), jnp.float32),
            pltpu.VMEM_SHARED((n_pad,), jnp.float32),
            [pltpu.SemaphoreType.DMA] * BLK,
            pltpu.SemaphoreType.DMA,
        ],
    )
    def hop_kernel(t0_h, t1_h, row_h, col_h, attr_h, zero_h, out_h,
                   rowb, colb, attrb, m0, m1, ts0, ts1, ac0, ac1, gsem, ssem):
        c = lax.axis_index("c")
        s = lax.axis_index("s")
        sl = pl.ds(s * rpt, rpt)
        pltpu.sync_copy(zero_h, ac0.at[sl])
        pltpu.sync_copy(zero_h, ac1.at[sl])
        pltpu.sync_copy(t0_h.at[sl], ts0.at[sl])
        pltpu.sync_copy(t1_h.at[sl], ts1.at[sl])
        plsc.subcore_barrier()
        base = (c * NS + s) * slabs * BLK

        def slab(i, carry):
            st = base + i * BLK
            pltpu.sync_copy(row_h.at[pl.ds(st, BLK)], rowb)
            pltpu.sync_copy(col_h.at[pl.ds(st, BLK)], colb)
            pltpu.sync_copy(attr_h.at[pl.ds(st, BLK)], attrb)
            gds = []
            for j in range(BLK):
                gds.append((
                    pltpu.async_copy(ts0.at[rowb.at[j]], m0.at[j], gsem[j]),
                    pltpu.async_copy(ts1.at[rowb.at[j]], m1.at[j], gsem[j]),
                ))
            sds = []
            for j in range(BLK):
                gds[j][0].wait()
                gds[j][1].wait()
                for v in range(CHUNK // L):
                    d = pl.ds(v * L, L)
                    a = attrb[j, d]
                    m0[j, d] = m0[j, d] * a
                    m1[j, d] = m1[j, d] * a
                sds.append(pltpu.async_copy(m0.at[j], ac0.at[colb.at[j]],
                                            ssem, add=True))
                sds.append(pltpu.async_copy(m1.at[j], ac1.at[colb.at[j]],
                                            ssem, add=True))
            for d in sds:
                d.wait()
            return carry

        lax.fori_loop(0, slabs, slab, 0)
        plsc.subcore_barrier()
        pltpu.sync_copy(ac0.at[sl], out_h.at[c * 2 + 0, sl])
        pltpu.sync_copy(ac1.at[sl], out_h.at[c * 2 + 1, sl])

    return hop_kernel


@functools.lru_cache(maxsize=None)
def _make_hop_rows(n_pad, nchunks):
    rpt = n_pad // NS
    f = L
    BLK = 8
    slabs = nchunks // BLK

    @functools.partial(
        pl.kernel,
        out_type=jax.ShapeDtypeStruct((NC, n_pad, f), jnp.float32),
        mesh=_mesh(),
        compiler_params=pltpu.CompilerParams(use_tc_tiling_on_sc=False),
        scratch_types=[
            pltpu.VMEM((BLK, CHUNK), jnp.int32),
            pltpu.VMEM((BLK, CHUNK), jnp.int32),
            pltpu.VMEM((BLK, CHUNK), jnp.float32),
            pltpu.VMEM((BLK, CHUNK, f), jnp.float32),
            pltpu.VMEM_SHARED((n_pad, f), jnp.float32),
            [pltpu.SemaphoreType.DMA] * BLK,
            pltpu.SemaphoreType.DMA,
        ],
    )
    def hop_kernel(t_h, row_h, col_h, attr_h, zero_h, out_h,
                   rowb, colb, attrb, msgb, acc, gsem, ssem):
        c = lax.axis_index("c")
        s = lax.axis_index("s")
        sl = pl.ds(s * rpt, rpt)
        pltpu.sync_copy(zero_h, acc.at[sl])
        plsc.subcore_barrier()
        base = (c * NS + s) * slabs * BLK
        dnums = lax.GatherDimensionNumbers(
            offset_dims=(), collapsed_slice_dims=(0,), start_index_map=(0,))

        def slab(i, carry):
            st = base + i * BLK
            pltpu.sync_copy(row_h.at[pl.ds(st, BLK)], rowb)
            pltpu.sync_copy(col_h.at[pl.ds(st, BLK)], colb)
            pltpu.sync_copy(attr_h.at[pl.ds(st, BLK)], attrb)
            gds = [
                pltpu.async_copy(t_h.at[rowb.at[j]], msgb.at[j], gsem[j])
                for j in range(BLK)
            ]
            sds = []
            for j in range(BLK):
                gds[j].wait()

                def grp(g, cg, j=j):
                    av = attrb[j, pl.ds(g * L, L)]
                    for t in range(L):
                        e = g * L + t
                        bc = lax.gather(
                            av, jnp.full((L, 1), t, jnp.int32), dnums, (1,),
                            mode=lax.GatherScatterMode.PROMISE_IN_BOUNDS)
                        msgb[j, e, :] = msgb[j, e, :] * bc
                    return cg

                lax.fori_loop(0, CHUNK // L, grp, 0)
                sds.append(pltpu.async_copy(msgb.at[j], acc.at[colb.at[j]],
                                            ssem, add=True))
            for d in sds:
                d.wait()
            return carry

        lax.fori_loop(0, slabs, slab, 0)
        plsc.subcore_barrier()
        pltpu.sync_copy(acc.at[sl], out_h.at[c, sl])

    return hop_kernel


def _row_spec(bf):
    return pl.BlockSpec((BN, bf), lambda i: (i, 0))


def _full_spec(shape):
    nd = len(shape)
    return pl.BlockSpec(shape, lambda i: (0,) * nd)


def _tc_prep(d0, d1):
    def body(d0r, d1r, o):
        deg = d0r[...] + d1r[...]
        o[...] = jnp.where(deg > 0, lax.rsqrt(jnp.maximum(deg, 1e-30)), 0.0)

    n_pad = d0.shape[0]
    return pl.pallas_call(
        body,
        grid=(n_pad // BN,),
        in_specs=[_row_spec(1), _row_spec(1)],
        out_specs=_row_spec(1),
        out_shape=jax.ShapeDtypeStruct((n_pad, 1), jnp.float32),
    )(d0, d1)


def _tc_init(x0, x1, dis, w, b):
    def body(x0r, x1r, dr, wr, br, acc_o, t0_o, t1_o):
        acc_o[...] = x0r[...] * wr[0:1, :] + x1r[...] * wr[1:2, :] + br[...]
        t0_o[...] = x0r[...] * dr[...]
        t1_o[...] = x1r[...] * dr[...]

    n_pad = x0.shape[0]
    o1 = jax.ShapeDtypeStruct((n_pad, 1), jnp.float32)
    return pl.pallas_call(
        body,
        grid=(n_pad // BN,),
        in_specs=[_row_spec(1), _row_spec(1), _row_spec(1),
                  _full_spec(w.shape), _full_spec(b.shape)],
        out_specs=[_row_spec(16), _row_spec(1), _row_spec(1)],
        out_shape=[jax.ShapeDtypeStruct((n_pad, 16), jnp.float32), o1, o1],
    )(x0, x1, dis, w, b)


def _tc_combine2(q00, q01, q10, q11, dis, acc, w, b):
    def body(a0, a1, b0, b1, dr, ar, wr, br, acc_o, t0_o, t1_o):
        d = dr[...]
        h0 = (a0[...] + b0[...]) * d
        h1 = (a1[...] + b1[...]) * d
        acc_o[...] = ar[...] + h0 * wr[0:1, :] + h1 * wr[1:2, :] + br[...]
        t0_o[...] = h0 * d
        t1_o[...] = h1 * d

    n_pad = q00.shape[0]
    o1 = jax.ShapeDtypeStruct((n_pad, 1), jnp.float32)
    return pl.pallas_call(
        body,
        grid=(n_pad // BN,),
        in_specs=[_row_spec(1)] * 5 + [_row_spec(16),
                  _full_spec(w.shape), _full_spec(b.shape)],
        out_specs=[_row_spec(16), _row_spec(1), _row_spec(1)],
        out_shape=[jax.ShapeDtypeStruct((n_pad, 16), jnp.float32), o1, o1],
    )(q00, q01, q10, q11, dis, acc, w, b)


def _tc_bridge(q00, q01, q10, q11, dis, acc, w14, b14, w20, b20):
    def body(a0, a1, b0, b1, dr, ar, w1r, b1r, w2r, b2r, acc_o, t_o):
        d = dr[...]
        h0 = (a0[...] + b0[...]) * d
        h1 = (a1[...] + b1[...]) * d
        a = jax.nn.relu(ar[...] + h0 * w1r[0:1, :] + h1 * w1r[1:2, :] + b1r[...])
        acc_o[...] = (
            jnp.dot(a, w2r[...], preferred_element_type=jnp.float32) + b2r[...]
        )
        t_o[...] = a * d

    n_pad = q00.shape[0]
    return pl.pallas_call(
        body,
        grid=(n_pad // BN,),
        in_specs=[_row_spec(1)] * 5 + [_row_spec(16),
                  _full_spec(w14.shape), _full_spec(b14.shape),
                  _full_spec(w20.shape), _full_spec(b20.shape)],
        out_specs=[_row_spec(16), _row_spec(16)],
        out_shape=[jax.ShapeDtypeStruct((n_pad, 16), jnp.float32),
                   jax.ShapeDtypeStruct((n_pad, 16), jnp.float32)],
    )(q00, q01, q10, q11, dis, acc, w14, b14, w20, b20)


def _tc_combine16(p0, p1, dis, acc, w, b):
    def body(p0r, p1r, dr, ar, wr, br, acc_o, t_o):
        d = dr[...]
        h = (p0r[...] + p1r[...]) * d
        acc_o[...] = (
            ar[...] + jnp.dot(h, wr[...], preferred_element_type=jnp.float32)
            + br[...]
        )
        t_o[...] = h * d

    n_pad = p0.shape[0]
    return pl.pallas_call(
        body,
        grid=(n_pad // BN,),
        in_specs=[_row_spec(16), _row_spec(16), _row_spec(1), _row_spec(16),
                  _full_spec(w.shape), _full_spec(b.shape)],
        out_specs=[_row_spec(16), _row_spec(16)],
        out_shape=[jax.ShapeDtypeStruct((n_pad, 16), jnp.float32),
                   jax.ShapeDtypeStruct((n_pad, 16), jnp.float32)],
    )(p0, p1, dis, acc, w, b)


def _tc_final(p0, p1, dis, acc, w24, b24, wend):
    def body(p0r, p1r, dr, ar, wr, br, wer, o):
        h = (p0r[...] + p1r[...]) * dr[...]
        a = jax.nn.relu(
            ar[...] + jnp.dot(h, wr[...], preferred_element_type=jnp.float32)
            + br[...]
        )
        o[...] = jax.nn.sigmoid(
            jnp.dot(a, wer[...], preferred_element_type=jnp.float32)
        )

    n_pad = p0.shape[0]
    return pl.pallas_call(
        body,
        grid=(n_pad // BN,),
        in_specs=[_row_spec(16), _row_spec(16), _row_spec(1), _row_spec(16),
                  _full_spec(w24.shape), _full_spec(b24.shape),
                  _full_spec(wend.shape)],
        out_specs=_row_spec(1),
        out_shape=jax.ShapeDtypeStruct((n_pad, 1), jnp.float32),
    )(p0, p1, dis, acc, w24, b24, wend)


def kernel(x, edge_index, edge_attr, batch, W1, b1, W2, b2, Wend):
    n = x.shape[0]
    e = edge_index.shape[1]
    kk = W1.shape[0]

    row = edge_index[0].astype(jnp.int32)
    col = edge_index[1].astype(jnp.int32)
    attr = edge_attr.astype(jnp.float32)

    epw = -(-e // (NW * CHUNK * BLK)) * CHUNK * BLK
    e_pad = NW * epw
    slabs = epw // (CHUNK * BLK)
    pad_n = e_pad - e
    pad_idx = (jnp.arange(pad_n, dtype=jnp.int32) * 37) % n
    row2 = jnp.concatenate([row, pad_idx]).reshape(-1, CHUNK)
    col2 = jnp.concatenate([col, pad_idx]).reshape(-1, CHUNK)
    attr2 = jnp.concatenate(
        [attr, jnp.zeros((pad_n,), jnp.float32)]).reshape(-1, CHUNK)

    n_pad = NS * (-(-n // (NS * CHUNK))) * CHUNK
    rpt = n_pad // NS
    x_p = jnp.zeros((n_pad, x.shape[1]), jnp.float32).at[:n].set(x)

    zero1 = jnp.zeros((rpt,), jnp.float32)
    zero16 = jnp.zeros((rpt, 16), jnp.float32)

    deg_p = _make_degree(n_pad, slabs)(col2, attr2, zero1)
    dis = _tc_prep(deg_p[0].reshape(n_pad, 1), deg_p[1].reshape(n_pad, 1))

    acc, t0, t1 = _tc_init(x_p[:, 0:1], x_p[:, 1:2], dis,
                           W1[0], b1[0].reshape(1, -1))
    hop2 = _make_hop_planar(n_pad, slabs)
    for k in range(1, kk):
        q = hop2(t0.reshape(n_pad), t1.reshape(n_pad),
                 row2, col2, attr2, zero1)
        planes = [q[i].reshape(n_pad, 1) for i in range(4)]
        if k < kk - 1:
            acc, t0, t1 = _tc_combine2(*planes, dis, acc,
                                       W1[k], b1[k].reshape(1, -1))
        else:
            acc, t = _tc_bridge(*planes, dis, acc,
                                W1[k], b1[k].reshape(1, -1),
                                W2[0], b2[0].reshape(1, -1))

    hop16 = _make_hop_rows(n_pad, slabs * BLK)
    for k in range(1, kk):
        p = hop16(t, row2, col2, attr2, zero16)
        if k < kk - 1:
            acc, t = _tc_combine16(p[0], p[1], dis, acc,
                                   W2[k], b2[k].reshape(1, -1))
        else:
            out = _tc_final(p[0], p[1], dis, acc,
                            W2[k], b2[k].reshape(1, -1), Wend)

    return out[:n]
